# baseline (speedup 1.0000x reference)
import functools

import jax
import jax.numpy as jnp
from jax import lax
from jax.experimental import pallas as pl
from jax.experimental.pallas import tpu as pltpu
from jax.experimental.pallas import tpu_sc as plsc

D = 64
DP = 128
NC, NS = 2, 16
NW = NC * NS
LANES = 16
BIG = 0x7FFFFFFF

RCHUNK = 160
GCHUNK = 160


def _relayout_call(table):
    V = table.shape[0]
    nchunk = -(-V // RCHUNK)
    iters = -(-nchunk // NW)
    vregs = RCHUNK * (D // LANES)

    mesh = plsc.VectorSubcoreMesh(
        core_axis_name="c", subcore_axis_name="s", num_cores=NC, num_subcores=NS
    )

    @functools.partial(
        pl.kernel,
        out_type=jax.ShapeDtypeStruct((V, DP), jnp.float32),
        mesh=mesh,
        scratch_types=[
            pltpu.VMEM((RCHUNK, D), jnp.float32),
            pltpu.VMEM((RCHUNK, D), jnp.float32),
            pltpu.VMEM((RCHUNK, DP), jnp.float32),
            pltpu.VMEM((RCHUNK, DP), jnp.float32),
            pltpu.SemaphoreType.DMA,
            pltpu.SemaphoreType.DMA,
            pltpu.SemaphoreType.DMA,
            pltpu.SemaphoreType.DMA,
        ],
    )
    def run(table_hbm, t2_hbm, in0, in1, w0, w1, s_r0, s_r1, s_w0, s_w1):
        wid = lax.axis_index("s") * NC + lax.axis_index("c")
        nin = (in0, in1)
        wide = (w0, w1)
        s_r = (s_r0, s_r1)
        s_w = (s_w0, s_w1)

        def chunk_of(i):
            return wid + i * NW

        def start_read(c, b):
            @pl.when(c < nchunk)
            def _():
                pltpu.async_copy(table_hbm.at[pl.ds(c * RCHUNK, RCHUNK)], nin[b], s_r[b])

        def wait_read(c, b):
            @pl.when(c < nchunk)
            def _():
                pltpu.make_async_copy(
                    table_hbm.at[pl.ds(0, RCHUNK)], nin[b], s_r[b]
                ).wait()

        def start_write(c, b):
            @pl.when(c < nchunk)
            def _():
                pltpu.async_copy(wide[b], t2_hbm.at[pl.ds(c * RCHUNK, RCHUNK)], s_w[b])

        def wait_write(c, b):
            @pl.when(c < nchunk)
            def _():
                pltpu.make_async_copy(
                    wide[b], t2_hbm.at[pl.ds(0, RCHUNK)], s_w[b]
                ).wait()

        def expand(c, b):
            @pl.when(c < nchunk)
            def _():
                def body(v, carry):
                    row = v >> 2
                    col = (v & 3) * LANES
                    wide[b][row, pl.ds(col, LANES)] = nin[b][row, pl.ds(col, LANES)]
                    return carry

                lax.fori_loop(0, vregs, body, 0)

        start_read(chunk_of(0), 0)
        start_read(chunk_of(1), 1)

        def outer(ii, carry):
            for b in (0, 1):
                i = ii * 2 + b
                c = chunk_of(i)
                wait_read(c, b)

                @pl.when(i >= 2)
                def _drain():
                    wait_write(chunk_of(i - 2), b)

                expand(c, b)
                start_write(c, b)
                start_read(chunk_of(i + 2), b)

            return carry

        assert iters % 2 == 0
        lax.fori_loop(0, iters // 2, outer, 0)
        wait_write(chunk_of(iters - 2), 0)
        wait_write(chunk_of(iters - 1), 1)

    return run(table)


def _gather_call(x, t2):
    NB, L = x.shape
    bpw = NB // NW
    nchunk = bpw
    nslice = -(-L // LANES)

    mesh = plsc.VectorSubcoreMesh(
        core_axis_name="c", subcore_axis_name="s", num_cores=NC, num_subcores=NS
    )

    @functools.partial(
        pl.kernel,
        out_type=jax.ShapeDtypeStruct((NB, L, D), jnp.float32),
        mesh=mesh,
        scratch_types=[
            pltpu.VMEM((L,), jnp.int32),
            pltpu.VMEM((L,), jnp.int32),
            pltpu.VMEM((L, DP), jnp.float32),
            pltpu.VMEM((L, DP), jnp.float32),
            pltpu.VMEM((1, L, D), jnp.float32),
            pltpu.VMEM((1, L, D), jnp.float32),
            pltpu.SemaphoreType.DMA,
            pltpu.SemaphoreType.DMA,
            pltpu.SemaphoreType.DMA,
            pltpu.SemaphoreType.DMA,
            pltpu.SemaphoreType.DMA,
            pltpu.SemaphoreType.DMA,
        ],
    )
    def run(x_hbm, t2_hbm, out_hbm, idxb0, idxb1, g0, g1, d0, d1,
            s_i0, s_i1, s_g0, s_g1, s_w0, s_w1):
        wid = lax.axis_index("s") * NC + lax.axis_index("c")
        base = wid * bpw
        idxb = (idxb0, idxb1)
        gbuf = (g0, g1)
        dbuf = (d0, d1)
        s_i = (s_i0, s_i1)
        s_g = (s_g0, s_g1)
        s_w = (s_w0, s_w1)

        def start_idx(g, b):
            pltpu.async_copy(x_hbm.at[base + g], idxb[b], s_i[b])

        def wait_idx(b):
            pltpu.make_async_copy(x_hbm.at[0], idxb[b], s_i[b]).wait()

        def start_gather(b):
            pltpu.async_copy(t2_hbm.at[idxb[b]], gbuf[b], s_g[b])

        def wait_gather(b):
            pltpu.make_async_copy(t2_hbm.at[idxb[b]], gbuf[b], s_g[b]).wait()

        def start_write(g, b):
            pltpu.async_copy(dbuf[b], out_hbm.at[pl.ds(base + g, 1)], s_w[b])

        def wait_write(b):
            pltpu.make_async_copy(dbuf[b], out_hbm.at[pl.ds(0, 1)], s_w[b]).wait()

        lane = lax.iota(jnp.int32, LANES)
        zero16 = jnp.zeros((LANES,), jnp.float32)

        def compact(b):
            def body(v, carry):
                row = v >> 2
                col = (v & 3) * LANES
                dbuf[b][0, row, pl.ds(col, LANES)] = gbuf[b][row, pl.ds(col, LANES)]
                return carry

            lax.fori_loop(0, L * (D // LANES), body, 0)

        def fix_padding_rows(b):
            def off_of(j):
                return jnp.minimum(j * LANES, L - LANES)

            def scan(j, vmin):
                off = off_of(j)
                v = idxb[b][pl.ds(off, LANES)]
                return jnp.minimum(vmin, jnp.where(v == 0, off + lane, BIG))

            vmin = lax.fori_loop(0, nslice, scan, jnp.full((LANES,), BIG, jnp.int32))
            fzp = vmin[0]
            for i in range(1, LANES):
                fzp = jnp.minimum(fzp, vmin[i])

            @pl.when(fzp != BIG)
            def _zero_rows():
                def body(j, carry):
                    off = off_of(j)
                    v = idxb[b][pl.ds(off, LANES)]
                    for i in range(LANES):
                        @pl.when(v[i] == 0)
                        def _clear():
                            row = off + i
                            for k in range(D // LANES):
                                dbuf[b][0, row, pl.ds(k * LANES, LANES)] = zero16

                    return carry

                lax.fori_loop(0, nslice, body, 0)

        start_idx(0, 0)
        start_idx(1, 1)
        wait_idx(0)
        start_gather(0)

        def outer(gg, carry):
            for b in (0, 1):
                g = gg * 2 + b
                wait_gather(b)

                @pl.when(g + 1 < nchunk)
                def _start_next_gather():
                    wait_idx(1 - b)
                    start_gather(1 - b)

                @pl.when(g >= 2)
                def _drain_prev_write():
                    wait_write(b)

                compact(b)
                fix_padding_rows(b)
                start_write(g, b)

                @pl.when(g + 2 < nchunk)
                def _start_next_idx():
                    start_idx(g + 2, b)

            return carry

        assert nchunk % 2 == 0
        lax.fori_loop(0, nchunk // 2, outer, 0)
        wait_write(0)
        wait_write(1)

    return run(x, t2)


def kernel(x, table):
    t2 = _relayout_call(table)
    return _gather_call(x.astype(jnp.int32), t2)


# speedup vs baseline: 1.2109x; 1.2109x over previous
import functools

import jax
import jax.numpy as jnp
from jax import lax
from jax.experimental import pallas as pl
from jax.experimental.pallas import tpu as pltpu
from jax.experimental.pallas import tpu_sc as plsc

D = 64
NC, NS = 2, 16
NW = NC * NS
CHUNK = 512
LANES = 16
BIG = 0x7FFFFFFF


def _embed_call(xf, table):
    B = xf.shape[0]
    bpw = B // NW
    nchunk = bpw // CHUNK
    vregs_per_chunk = CHUNK // LANES

    mesh = plsc.VectorSubcoreMesh(
        core_axis_name="c", subcore_axis_name="s", num_cores=NC, num_subcores=NS
    )

    @functools.partial(
        pl.kernel,
        out_type=jax.ShapeDtypeStruct((B, D), jnp.float32),
        mesh=mesh,
        compiler_params=pltpu.CompilerParams(use_tc_tiling_on_sc=False),
        scratch_types=[
            pltpu.VMEM((CHUNK,), jnp.int32),
            pltpu.VMEM((CHUNK,), jnp.int32),
            pltpu.VMEM((CHUNK, D), jnp.float32),
            pltpu.VMEM((CHUNK, D), jnp.float32),
            pltpu.SemaphoreType.DMA,
            pltpu.SemaphoreType.DMA,
            pltpu.SemaphoreType.DMA,
            pltpu.SemaphoreType.DMA,
            pltpu.SemaphoreType.DMA,
            pltpu.SemaphoreType.DMA,
        ],
    )
    def run(x_hbm, table_hbm, out_hbm, idxb0, idxb1, data0, data1,
            s_i0, s_i1, s_g0, s_g1, s_w0, s_w1):
        wid = lax.axis_index("s") * NC + lax.axis_index("c")
        base = wid * bpw
        idxb = (idxb0, idxb1)
        data = (data0, data1)
        s_i = (s_i0, s_i1)
        s_g = (s_g0, s_g1)
        s_w = (s_w0, s_w1)

        def start_idx(g, b):
            pltpu.async_copy(x_hbm.at[pl.ds(base + g * CHUNK, CHUNK)], idxb[b], s_i[b])

        def wait_idx(b):
            pltpu.make_async_copy(x_hbm.at[pl.ds(0, CHUNK)], idxb[b], s_i[b]).wait()

        def start_gather(b):
            pltpu.async_copy(table_hbm.at[idxb[b]], data[b], s_g[b])

        def wait_gather(b):
            pltpu.make_async_copy(table_hbm.at[idxb[b]], data[b], s_g[b]).wait()

        def start_write(g, b):
            pltpu.async_copy(
                data[b], out_hbm.at[pl.ds(base + g * CHUNK, CHUNK)], s_w[b]
            )

        def wait_write(b):
            pltpu.make_async_copy(
                data[b], out_hbm.at[pl.ds(0, CHUNK)], s_w[b]
            ).wait()

        lane = lax.iota(jnp.int32, LANES)
        zero16 = jnp.zeros((LANES,), jnp.float32)

        def fix_padding_rows(b):
            def scan(j, vmin):
                v = idxb[b][pl.ds(j * LANES, LANES)]
                return jnp.minimum(vmin, jnp.where(v == 0, j * LANES + lane, BIG))

            vmin = lax.fori_loop(
                0, vregs_per_chunk, scan, jnp.full((LANES,), BIG, jnp.int32)
            )
            fzp = vmin[0]
            for i in range(1, LANES):
                fzp = jnp.minimum(fzp, vmin[i])

            @pl.when(fzp != BIG)
            def _zero_rows():
                def body(j, carry):
                    v = idxb[b][pl.ds(j * LANES, LANES)]
                    for i in range(LANES):
                        @pl.when(v[i] == 0)
                        def _clear():
                            row = j * LANES + i
                            for k in range(D // LANES):
                                data[b][row, pl.ds(k * LANES, LANES)] = zero16

                    return carry

                lax.fori_loop(0, vregs_per_chunk, body, 0)

        start_idx(0, 0)
        start_idx(1, 1)
        wait_idx(0)
        start_gather(0)

        def outer(gg, carry):
            for b in (0, 1):
                g = gg * 2 + b
                wait_gather(b)
                fix_padding_rows(b)

                @pl.when(g >= 1)
                def _drain_prev_write():
                    wait_write(1 - b)

                start_write(g, b)

                @pl.when(g + 1 < nchunk)
                def _start_next_gather():
                    wait_idx(1 - b)
                    start_gather(1 - b)

                @pl.when(g + 2 < nchunk)
                def _start_next_idx():
                    start_idx(g + 2, b)

            return carry

        assert nchunk % 2 == 0
        lax.fori_loop(0, nchunk // 2, outer, 0)
        wait_write((nchunk - 1) % 2)

    return run(xf, table)


def kernel(x, table):
    b, l = x.shape
    xf = x.reshape(-1).astype(jnp.int32)
    out = _embed_call(xf, table)
    return out.reshape(b, l, D)


# speedup vs baseline: 1.2158x; 1.0040x over previous
import functools

import jax
import jax.numpy as jnp
from jax import lax
from jax.experimental import pallas as pl
from jax.experimental.pallas import tpu as pltpu
from jax.experimental.pallas import tpu_sc as plsc

D = 64
NC, NS = 2, 16
NW = NC * NS
CHUNK = 800
LANES = 16
BIG = 0x7FFFFFFF


def _embed_call(xf, table):
    B = xf.shape[0]
    bpw = B // NW
    nchunk = bpw // CHUNK
    vregs_per_chunk = CHUNK // LANES

    mesh = plsc.VectorSubcoreMesh(
        core_axis_name="c", subcore_axis_name="s", num_cores=NC, num_subcores=NS
    )

    @functools.partial(
        pl.kernel,
        out_type=jax.ShapeDtypeStruct((B, D), jnp.float32),
        mesh=mesh,
        compiler_params=pltpu.CompilerParams(use_tc_tiling_on_sc=False),
        scratch_types=[
            pltpu.VMEM((CHUNK,), jnp.int32),
            pltpu.VMEM((CHUNK,), jnp.int32),
            pltpu.VMEM((CHUNK, D), jnp.float32),
            pltpu.VMEM((CHUNK, D), jnp.float32),
            pltpu.SemaphoreType.DMA,
            pltpu.SemaphoreType.DMA,
            pltpu.SemaphoreType.DMA,
            pltpu.SemaphoreType.DMA,
            pltpu.SemaphoreType.DMA,
            pltpu.SemaphoreType.DMA,
        ],
    )
    def run(x_hbm, table_hbm, out_hbm, idxb0, idxb1, data0, data1,
            s_i0, s_i1, s_g0, s_g1, s_w0, s_w1):
        wid = lax.axis_index("s") * NC + lax.axis_index("c")
        base = wid * bpw
        idxb = (idxb0, idxb1)
        data = (data0, data1)
        s_i = (s_i0, s_i1)
        s_g = (s_g0, s_g1)
        s_w = (s_w0, s_w1)

        def start_idx(g, b):
            pltpu.async_copy(x_hbm.at[pl.ds(base + g * CHUNK, CHUNK)], idxb[b], s_i[b])

        def wait_idx(b):
            pltpu.make_async_copy(x_hbm.at[pl.ds(0, CHUNK)], idxb[b], s_i[b]).wait()

        def start_gather(b):
            pltpu.async_copy(table_hbm.at[idxb[b]], data[b], s_g[b])

        def wait_gather(b):
            pltpu.make_async_copy(table_hbm.at[idxb[b]], data[b], s_g[b]).wait()

        def start_write(g, b):
            pltpu.async_copy(
                data[b], out_hbm.at[pl.ds(base + g * CHUNK, CHUNK)], s_w[b]
            )

        def wait_write(b):
            pltpu.make_async_copy(
                data[b], out_hbm.at[pl.ds(0, CHUNK)], s_w[b]
            ).wait()

        lane = lax.iota(jnp.int32, LANES)
        zero16 = jnp.zeros((LANES,), jnp.float32)

        def fix_padding_rows(b):
            def scan(j, vmin):
                v = idxb[b][pl.ds(j * LANES, LANES)]
                return jnp.minimum(vmin, jnp.where(v == 0, j * LANES + lane, BIG))

            vmin = lax.fori_loop(
                0, vregs_per_chunk, scan, jnp.full((LANES,), BIG, jnp.int32)
            )
            fzp = vmin[0]
            for i in range(1, LANES):
                fzp = jnp.minimum(fzp, vmin[i])

            @pl.when(fzp != BIG)
            def _zero_rows():
                def body(j, carry):
                    v = idxb[b][pl.ds(j * LANES, LANES)]
                    for i in range(LANES):
                        @pl.when(v[i] == 0)
                        def _clear():
                            row = j * LANES + i
                            for k in range(D // LANES):
                                data[b][row, pl.ds(k * LANES, LANES)] = zero16

                    return carry

                lax.fori_loop(0, vregs_per_chunk, body, 0)

        start_idx(0, 0)
        start_idx(1, 1)
        wait_idx(0)
        start_gather(0)

        def outer(gg, carry):
            for b in (0, 1):
                g = gg * 2 + b
                wait_gather(b)
                fix_padding_rows(b)

                @pl.when(g >= 1)
                def _drain_prev_write():
                    wait_write(1 - b)

                start_write(g, b)

                @pl.when(g + 1 < nchunk)
                def _start_next_gather():
                    wait_idx(1 - b)
                    start_gather(1 - b)

                @pl.when(g + 2 < nchunk)
                def _start_next_idx():
                    start_idx(g + 2, b)

            return carry

        assert nchunk % 2 == 0
        lax.fori_loop(0, nchunk // 2, outer, 0)
        wait_write((nchunk - 1) % 2)

    return run(xf, table)


def kernel(x, table):
    b, l = x.shape
    xf = x.reshape(-1).astype(jnp.int32)
    out = _embed_call(xf, table)
    return out.reshape(b, l, D)
